# baseline (device time: 47254 ns/iter reference)
import jax
import jax.numpy as jnp
from jax import lax
from jax.experimental import pallas as pl
from jax.experimental.pallas import tpu as pltpu

N_DEV = 8
M = 1536
N = 1536
K = 768
P_PARTS = 12
PART_ROWS = M // P_PARTS
MASK_X, MASK_Y, MASK_Z = 1, 3, 4
ORDER = [
    [MASK_X, MASK_Y, MASK_Z],
    [MASK_Y, MASK_Z, MASK_X],
    [MASK_Z, MASK_X, MASK_Y],
] * 4
EX = [PART_ROWS // 2, PART_ROWS // 4, PART_ROWS // 4]
SCR_OFF = [0, EX[0], EX[0] + EX[1]]
SCR_ROWS = EX[0] + EX[1] + EX[2]


def _side_bit(my, mask):
    if mask == MASK_X:
        return (my ^ (my >> 1)) & 1
    if mask == MASK_Y:
        return (my >> 1) & 1
    return (my >> 2) & 1


def kernel(A, B):
    def body(a_ref, b_ref, out_ref, abf_ref, bbf_ref, p_ref, stage_ref,
             scr_ref, ag_ref, rs_send, rs_recv, ag_send, ag_recv):
        my = lax.axis_index("i")

        barrier = pltpu.get_barrier_semaphore()
        for mask in (MASK_X, MASK_Y, MASK_Z):
            pl.semaphore_signal(barrier, inc=1, device_id=(my ^ mask,),
                                device_id_type=pl.DeviceIdType.MESH)

        abf_ref[...] = a_ref[...].astype(jnp.bfloat16)
        bbf_ref[...] = b_ref[...].astype(jnp.bfloat16)

        def mm(row0, rows):
            p_ref[pl.ds(row0, rows), :] = lax.dot_general(
                abf_ref[pl.ds(row0, rows), :], bbf_ref[...],
                (((1,), (0,)), ((), ())),
                preferred_element_type=jnp.float32)

        def make_rs(part, s, send_lo):
            ex = EX[s]
            sb = part * SCR_ROWS + SCR_OFF[s]
            stage_ref[pl.ds(sb, ex), :] = (
                p_ref[pl.ds(part * PART_ROWS + send_lo, ex), :]
                .astype(jnp.bfloat16))
            return pltpu.make_async_remote_copy(
                src_ref=stage_ref.at[pl.ds(sb, ex), :],
                dst_ref=scr_ref.at[pl.ds(sb, ex), :],
                send_sem=rs_send.at[part, s],
                recv_sem=rs_recv.at[part, s],
                device_id=(my ^ ORDER[part][s],),
                device_id_type=pl.DeviceIdType.MESH,
            ), sb

        def make_ag(part, k, g, rows):
            return pltpu.make_async_remote_copy(
                src_ref=ag_ref.at[pl.ds(g, rows), :],
                dst_ref=ag_ref.at[pl.ds(g, rows), :],
                send_sem=ag_send.at[part, k],
                recv_sem=ag_recv.at[part, k],
                device_id=(my ^ ORDER[part][1 - k],),
                device_id_type=pl.DeviceIdType.MESH,
            )

        lo = [0] * P_PARTS
        rs = [[None] * 3 for _ in range(P_PARTS)]
        for part in range(P_PARTS):
            bbit = _side_bit(my, ORDER[part][0])
            send_lo = (1 - bbit) * EX[0]
            mm(part * PART_ROWS + send_lo, EX[0])
            rs[part][0] = make_rs(part, 0, send_lo)
            if part == 0:
                pl.semaphore_wait(barrier, 3)
            rs[part][0][0].start()
            lo[part] = bbit * EX[0]
        for part in range(P_PARTS):
            mm(part * PART_ROWS + lo[part], EX[0])

        ag = [[None] * 2 for _ in range(P_PARTS)]
        for s in range(3):
            ex = EX[s]
            for part in range(P_PARTS):
                rdma, sb = rs[part][s]
                rdma.wait()
                g = part * PART_ROWS + lo[part]
                p_ref[pl.ds(g, ex), :] = (
                    p_ref[pl.ds(g, ex), :]
                    + scr_ref[pl.ds(sb, ex), :].astype(jnp.float32))
                if s == 0:
                    nbit = _side_bit(my, ORDER[part][1])
                    rs[part][1] = make_rs(
                        part, 1, lo[part] + (1 - nbit) * EX[1])
                    rs[part][1][0].start()
                    lo[part] = lo[part] + nbit * EX[1]
                elif s == 1:
                    rs[part][2] = make_rs(part, 2, lo[part])
                    rs[part][2][0].start()
                else:
                    z = p_ref[pl.ds(g, ex), :]
                    act = z / (1.0 + jnp.exp(-z))
                    out_ref[pl.ds(g, ex), :] = act
                    ag_ref[pl.ds(g, ex), :] = act.astype(jnp.bfloat16)
                    ag[part][0] = make_ag(part, 0, g, ex)
                    ag[part][0].start()

        own = EX[2]
        for k in range(2):
            for part in range(P_PARTS):
                ag[part][k].wait()
                bbit = _side_bit(my, ORDER[part][1 - k])
                new_lo = lo[part] - bbit * own
                recv_lo = new_lo + (1 - bbit) * own
                lo[part] = new_lo
                if k == 0:
                    ag[part][1] = make_ag(
                        part, 1, part * PART_ROWS + new_lo, 2 * own)
                    ag[part][1].start()
                g = part * PART_ROWS + recv_lo
                out_ref[pl.ds(g, own), :] = (
                    ag_ref[pl.ds(g, own), :].astype(jnp.float32))
            own *= 2

    return pl.pallas_call(
        body,
        out_shape=jax.ShapeDtypeStruct((M, N), jnp.float32),
        in_specs=[pl.BlockSpec(memory_space=pltpu.VMEM),
                  pl.BlockSpec(memory_space=pltpu.VMEM)],
        out_specs=pl.BlockSpec(memory_space=pltpu.VMEM),
        scratch_shapes=[
            pltpu.VMEM((M, K), jnp.bfloat16),
            pltpu.VMEM((K, N), jnp.bfloat16),
            pltpu.VMEM((M, N), jnp.float32),
            pltpu.VMEM((P_PARTS * SCR_ROWS, N), jnp.bfloat16),
            pltpu.VMEM((P_PARTS * SCR_ROWS, N), jnp.bfloat16),
            pltpu.VMEM((M, N), jnp.bfloat16),
            pltpu.SemaphoreType.DMA((P_PARTS, 3)),
            pltpu.SemaphoreType.DMA((P_PARTS, 3)),
            pltpu.SemaphoreType.DMA((P_PARTS, 2)),
            pltpu.SemaphoreType.DMA((P_PARTS, 2)),
        ],
        compiler_params=pltpu.CompilerParams(collective_id=0),
    )(A, B)


# device time: 45337 ns/iter; 1.0423x vs baseline; 1.0423x over previous
import jax
import jax.numpy as jnp
from jax import lax
from jax.experimental import pallas as pl
from jax.experimental.pallas import tpu as pltpu

N_DEV = 8
M = 1536
N = 1536
K = 768
P_PARTS = 6
PART_ROWS = M // P_PARTS
MASK_X, MASK_Y, MASK_Z = 1, 3, 4
ORDER = [
    [MASK_X, MASK_Y, MASK_Z],
    [MASK_Y, MASK_Z, MASK_X],
    [MASK_Z, MASK_X, MASK_Y],
] * 2
EX = [PART_ROWS // 2, PART_ROWS // 4, PART_ROWS // 4]
SCR_OFF = [0, EX[0], EX[0] + EX[1]]
SCR_ROWS = EX[0] + EX[1] + EX[2]


def _side_bit(my, mask):
    if mask == MASK_X:
        return (my ^ (my >> 1)) & 1
    if mask == MASK_Y:
        return (my >> 1) & 1
    return (my >> 2) & 1


def kernel(A, B):
    def body(a_ref, b_ref, out_ref, abf_ref, bbf_ref, p_ref, stage_ref,
             scr_ref, ag_ref, rs_send, rs_recv, ag_send, ag_recv):
        my = lax.axis_index("i")

        barrier = pltpu.get_barrier_semaphore()
        for mask in (MASK_X, MASK_Y, MASK_Z):
            pl.semaphore_signal(barrier, inc=1, device_id=(my ^ mask,),
                                device_id_type=pl.DeviceIdType.MESH)

        abf_ref[...] = a_ref[...].astype(jnp.bfloat16)
        bbf_ref[...] = b_ref[...].astype(jnp.bfloat16)

        def mm(row0, rows):
            p_ref[pl.ds(row0, rows), :] = lax.dot_general(
                abf_ref[pl.ds(row0, rows), :], bbf_ref[...],
                (((1,), (0,)), ((), ())),
                preferred_element_type=jnp.float32)

        def make_rs(part, s, send_lo):
            ex = EX[s]
            sb = part * SCR_ROWS + SCR_OFF[s]
            stage_ref[pl.ds(sb, ex), :] = (
                p_ref[pl.ds(part * PART_ROWS + send_lo, ex), :]
                .astype(jnp.bfloat16))
            return pltpu.make_async_remote_copy(
                src_ref=stage_ref.at[pl.ds(sb, ex), :],
                dst_ref=scr_ref.at[pl.ds(sb, ex), :],
                send_sem=rs_send.at[part, s],
                recv_sem=rs_recv.at[part, s],
                device_id=(my ^ ORDER[part][s],),
                device_id_type=pl.DeviceIdType.MESH,
            ), sb

        def make_ag(part, k, g, rows):
            return pltpu.make_async_remote_copy(
                src_ref=ag_ref.at[pl.ds(g, rows), :],
                dst_ref=ag_ref.at[pl.ds(g, rows), :],
                send_sem=ag_send.at[part, k],
                recv_sem=ag_recv.at[part, k],
                device_id=(my ^ ORDER[part][1 - k],),
                device_id_type=pl.DeviceIdType.MESH,
            )

        lo = [0] * P_PARTS
        rs = [[None] * 3 for _ in range(P_PARTS)]
        for part in range(P_PARTS):
            bbit = _side_bit(my, ORDER[part][0])
            send_lo = (1 - bbit) * EX[0]
            mm(part * PART_ROWS + send_lo, EX[0])
            rs[part][0] = make_rs(part, 0, send_lo)
            if part == 0:
                pl.semaphore_wait(barrier, 3)
            rs[part][0][0].start()
            lo[part] = bbit * EX[0]
        for part in range(P_PARTS):
            mm(part * PART_ROWS + lo[part], EX[0])

        ag = [[None] * 2 for _ in range(P_PARTS)]
        for s in range(3):
            ex = EX[s]
            for part in range(P_PARTS):
                rdma, sb = rs[part][s]
                rdma.wait()
                g = part * PART_ROWS + lo[part]
                p_ref[pl.ds(g, ex), :] = (
                    p_ref[pl.ds(g, ex), :]
                    + scr_ref[pl.ds(sb, ex), :].astype(jnp.float32))
                if s == 0:
                    nbit = _side_bit(my, ORDER[part][1])
                    rs[part][1] = make_rs(
                        part, 1, lo[part] + (1 - nbit) * EX[1])
                    rs[part][1][0].start()
                    lo[part] = lo[part] + nbit * EX[1]
                elif s == 1:
                    rs[part][2] = make_rs(part, 2, lo[part])
                    rs[part][2][0].start()
                else:
                    z = p_ref[pl.ds(g, ex), :]
                    act = z / (1.0 + jnp.exp(-z))
                    out_ref[pl.ds(g, ex), :] = act
                    ag_ref[pl.ds(g, ex), :] = act.astype(jnp.bfloat16)
                    ag[part][0] = make_ag(part, 0, g, ex)
                    ag[part][0].start()

        own = EX[2]
        for k in range(2):
            for part in range(P_PARTS):
                ag[part][k].wait()
                bbit = _side_bit(my, ORDER[part][1 - k])
                new_lo = lo[part] - bbit * own
                recv_lo = new_lo + (1 - bbit) * own
                lo[part] = new_lo
                if k == 0:
                    ag[part][1] = make_ag(
                        part, 1, part * PART_ROWS + new_lo, 2 * own)
                    ag[part][1].start()
                g = part * PART_ROWS + recv_lo
                out_ref[pl.ds(g, own), :] = (
                    ag_ref[pl.ds(g, own), :].astype(jnp.float32))
            own *= 2

    return pl.pallas_call(
        body,
        out_shape=jax.ShapeDtypeStruct((M, N), jnp.float32),
        in_specs=[pl.BlockSpec(memory_space=pltpu.VMEM),
                  pl.BlockSpec(memory_space=pltpu.VMEM)],
        out_specs=pl.BlockSpec(memory_space=pltpu.VMEM),
        scratch_shapes=[
            pltpu.VMEM((M, K), jnp.bfloat16),
            pltpu.VMEM((K, N), jnp.bfloat16),
            pltpu.VMEM((M, N), jnp.float32),
            pltpu.VMEM((P_PARTS * SCR_ROWS, N), jnp.bfloat16),
            pltpu.VMEM((P_PARTS * SCR_ROWS, N), jnp.bfloat16),
            pltpu.VMEM((M, N), jnp.bfloat16),
            pltpu.SemaphoreType.DMA((P_PARTS, 3)),
            pltpu.SemaphoreType.DMA((P_PARTS, 3)),
            pltpu.SemaphoreType.DMA((P_PARTS, 2)),
            pltpu.SemaphoreType.DMA((P_PARTS, 2)),
        ],
        compiler_params=pltpu.CompilerParams(collective_id=0),
    )(A, B)
